# Initial kernel scaffold; baseline (speedup 1.0000x reference)
import jax
import jax.numpy as jnp
from jax.experimental import pallas as pl


def kernel(base, proj_weight, proj_bias):
    raise NotImplementedError("write your pallas kernel here")


# speedup vs baseline: 1.4691x; 1.4691x over previous
import jax
import jax.numpy as jnp
from jax.experimental import pallas as pl
from jax.experimental.pallas import tpu as pltpu

B, S, D = 4, 4096, 4096
TR = 512


def _body(w_ref, bias_ref, x_ref, out_ref, lat_ref):
    x = x_ref[...]
    w = w_ref[...]
    out_ref[...] = x + w
    acc = jnp.sum(x * w, axis=1) + bias_ref[0]
    lat_ref[0, 0, :] = jnp.maximum(acc, 0.0)


def kernel(base, proj_weight, proj_bias):
    rows = B * S
    n_tiles = rows // TR
    x2 = base.reshape(rows, D)
    out2, lat2 = pl.pallas_call(
        _body,
        grid=(n_tiles,),
        in_specs=[
            pl.BlockSpec((1, D), lambda i: (0, 0)),
            pl.BlockSpec(memory_space=pltpu.SMEM),
            pl.BlockSpec((TR, D), lambda i: (i, 0)),
        ],
        out_specs=[
            pl.BlockSpec((TR, D), lambda i: (i, 0)),
            pl.BlockSpec((1, 1, TR), lambda i: (i, 0, 0)),
        ],
        out_shape=[
            jax.ShapeDtypeStruct((rows, D), base.dtype),
            jax.ShapeDtypeStruct((n_tiles, 1, TR), jnp.float32),
        ],
    )(proj_weight, proj_bias, x2)
    return out2.reshape(B, S, D), lat2.reshape(B, S)
